# Initial kernel scaffold; baseline (speedup 1.0000x reference)
import jax
import jax.numpy as jnp
from jax.experimental import pallas as pl


def kernel(x, W_router, fc1, fc2):
    raise NotImplementedError("write your pallas kernel here")


# speedup vs baseline: 2.6603x; 2.6603x over previous
import functools
import math

import jax
import jax.numpy as jnp
from jax.experimental import pallas as pl
from jax.experimental.pallas import tpu as pltpu

_TOP_K = 2
_CAP_FACTOR = 1.25


def _ffn_body(x_ref, w1_ref, w2_ref, o_ref):
    j = pl.program_id(1)
    xb = x_ref[0]
    w1 = w1_ref[0]
    w2 = w2_ref[0]
    h = jax.lax.dot_general(xb, w1, (((1,), (1,)), ((), ())),
                            preferred_element_type=jnp.float32)
    h = 0.5 * h * (1.0 + jax.lax.erf(h * 0.7071067811865476))
    y = jax.lax.dot_general(h, w2, (((1,), (1,)), ((), ())),
                            preferred_element_type=jnp.float32)

    @pl.when(j == 0)
    def _init():
        o_ref[0] = y

    @pl.when(j != 0)
    def _acc():
        o_ref[0] += y


def _expert_ffn(xbuf, fc1, fc2, block_ff=1024):
    E, cap, D = xbuf.shape
    DFF = fc1.shape[1]
    block_ff = min(block_ff, DFF)
    nj = DFF // block_ff
    return pl.pallas_call(
        _ffn_body,
        grid=(E, nj),
        in_specs=[
            pl.BlockSpec((1, cap, D), lambda e, j: (e, 0, 0)),
            pl.BlockSpec((1, block_ff, D), lambda e, j: (e, j, 0)),
            pl.BlockSpec((1, D, block_ff), lambda e, j: (e, 0, j)),
        ],
        out_specs=pl.BlockSpec((1, cap, D), lambda e, j: (e, 0, 0)),
        out_shape=jax.ShapeDtypeStruct((E, cap, D), jnp.float32),
        compiler_params=pltpu.CompilerParams(
            dimension_semantics=("arbitrary", "arbitrary")),
    )(xbuf, fc1, fc2)


def kernel(x, W_router, fc1, fc2):
    T, D = x.shape
    E = W_router.shape[0]
    S = T * _TOP_K
    cap = max(1, math.ceil(S * _CAP_FACTOR / E))

    logits = x @ W_router.T
    top_v, top_i = jax.lax.top_k(logits, _TOP_K)
    m = jnp.max(top_v, axis=-1, keepdims=True)
    ev = jnp.exp(top_v - m)
    top_p = ev / (ev.sum(axis=-1, keepdims=True) + 1e-12)

    e_flat = top_i.reshape(-1)
    p_flat = top_p.reshape(-1)
    onehot = (e_flat[:, None] == jnp.arange(E)[None, :]).astype(jnp.int32)
    excl = jnp.cumsum(onehot, axis=0) - onehot
    rank = jnp.take_along_axis(excl, e_flat[:, None], axis=1)[:, 0]
    kept = rank < cap
    slot = e_flat * cap + rank
    tok = jnp.repeat(jnp.arange(T, dtype=jnp.int32), _TOP_K)

    slot_sc = jnp.where(kept, slot, E * cap)
    src = jnp.zeros((E * cap,), jnp.int32).at[slot_sc].set(tok, mode='drop')
    xbuf = x[src].reshape(E, cap, D)

    y = _expert_ffn(xbuf, fc1, fc2).reshape(E * cap, D)

    w = jnp.where(kept, p_flat, 0.0)
    slot_g = jnp.where(kept, slot, 0)
    gath = y[slot_g] * w[:, None]
    return gath.reshape(T, _TOP_K, D).sum(axis=1)
